# Initial kernel scaffold; baseline (speedup 1.0000x reference)
import jax
import jax.numpy as jnp
from jax.experimental import pallas as pl


def kernel(x, edge_index, self_kernel, self_bias, aggr_self_kernel, aggr_self_bias, aggr_neighbor_kernel):
    raise NotImplementedError("write your pallas kernel here")


# speedup vs baseline: 2.5609x; 2.5609x over previous
import functools

import jax
import jax.numpy as jnp
from jax import lax
from jax.experimental import pallas as pl
from jax.experimental.pallas import tpu as pltpu
from jax.experimental.pallas import tpu_sc as plsc

N = 10000
E = 160000
D = 256
DH = 128

NC = 2
NS = 16
CH = 64
EPT = 10240
E_PAD = EPT * NS
NCHUNK = EPT // CH
EPW = E_PAD // (NC * NS)
GROWS = 10240
ZR = GROWS // NS
ZB = 80


def _sc_scatter(x_lo, x_hi, rowp, colp):
    mesh = plsc.VectorSubcoreMesh(core_axis_name="c", subcore_axis_name="s")

    @functools.partial(
        pl.kernel,
        out_type=(
            jax.ShapeDtypeStruct((GROWS, DH), jnp.float32),
            jax.ShapeDtypeStruct((GROWS, DH), jnp.float32),
        ),
        mesh=mesh,
        scratch_types=[
            pltpu.VMEM((CH,), jnp.int32),
            pltpu.VMEM((CH,), jnp.int32),
            pltpu.VMEM((CH, DH), jnp.float32),
            pltpu.SemaphoreType.DMA,
            pltpu.VMEM((ZB, DH), jnp.float32),
            pltpu.VMEM_SHARED((GROWS, DH), jnp.float32),
        ],
    )
    def k(xlo_hbm, xhi_hbm, row_hbm, col_hbm, g0_hbm, g1_hbm,
          rowv, colv, buf, sem, zbuf, g_sh):
        c = lax.axis_index("c")
        s = lax.axis_index("s")

        zrow = jnp.zeros((16,), jnp.float32)

        @pl.loop(0, ZB)
        def _(i):
            @pl.loop(0, DH // 16)
            def _(j):
                zbuf[i, pl.ds(j * 16, 16)] = zrow

        @pl.loop(0, ZR // ZB)
        def _(q):
            pltpu.sync_copy(zbuf, g_sh.at[pl.ds(s * ZR + q * ZB, ZB)])

        plsc.subcore_barrier()

        base = s * EPT

        def edge_loop(x_ref):
            @pl.loop(0, NCHUNK)
            def _(kk):
                off = base + kk * CH
                pltpu.sync_copy(row_hbm.at[pl.ds(off, CH)], rowv)
                pltpu.sync_copy(col_hbm.at[pl.ds(off, CH)], colv)
                pltpu.async_copy(x_ref.at[colv], buf, sem).wait()
                pltpu.sync_copy(buf, g_sh.at[rowv], add=True)

        @pl.when(c == 0)
        def _():
            edge_loop(xlo_hbm)

        @pl.when(c == 1)
        def _():
            edge_loop(xhi_hbm)

        plsc.subcore_barrier()

        @pl.when(c == 0)
        def _():
            pltpu.sync_copy(g_sh.at[pl.ds(s * ZR, ZR)],
                            g0_hbm.at[pl.ds(s * ZR, ZR)])

        @pl.when(c == 1)
        def _():
            pltpu.sync_copy(g_sh.at[pl.ds(s * ZR, ZR)],
                            g1_hbm.at[pl.ds(s * ZR, ZR)])

    return k(x_lo, x_hi, rowp, colp)


def _sc_degree(rowp, dep):
    mesh = plsc.VectorSubcoreMesh(core_axis_name="c", subcore_axis_name="s")

    @functools.partial(
        pl.kernel,
        out_type=(
            jax.ShapeDtypeStruct((GROWS, DH), jnp.float32),
            jax.ShapeDtypeStruct((GROWS, DH), jnp.float32),
        ),
        mesh=mesh,
        scratch_types=[
            pltpu.VMEM((CH,), jnp.int32),
            pltpu.VMEM((CH, DH), jnp.float32),
            pltpu.VMEM((ZB, DH), jnp.float32),
            pltpu.VMEM_SHARED((GROWS, DH), jnp.float32),
        ],
    )
    def k(row_hbm, dep_hbm, d0_hbm, d1_hbm, rowv, onesb, zbuf, d_sh):
        del dep_hbm
        c = lax.axis_index("c")
        s = lax.axis_index("s")

        zrow = jnp.zeros((16,), jnp.float32)
        e0row = jnp.where(lax.iota(jnp.int32, 16) == 0, 1.0, 0.0)

        @pl.loop(0, ZB)
        def _(i):
            @pl.loop(0, DH // 16)
            def _(j):
                zbuf[i, pl.ds(j * 16, 16)] = zrow

        @pl.loop(0, CH)
        def _(i):
            onesb[i, pl.ds(0, 16)] = e0row

            @pl.loop(1, DH // 16)
            def _(j):
                onesb[i, pl.ds(j * 16, 16)] = zrow

        @pl.loop(0, ZR // ZB)
        def _(q):
            pltpu.sync_copy(zbuf, d_sh.at[pl.ds(s * ZR + q * ZB, ZB)])

        plsc.subcore_barrier()

        base = (c * NS + s) * EPW

        @pl.loop(0, EPW // CH)
        def _(kk):
            off = base + kk * CH
            pltpu.sync_copy(row_hbm.at[pl.ds(off, CH)], rowv)
            pltpu.sync_copy(onesb, d_sh.at[rowv], add=True)

        plsc.subcore_barrier()

        @pl.when(c == 0)
        def _():
            pltpu.sync_copy(d_sh.at[pl.ds(s * ZR, ZR)],
                            d0_hbm.at[pl.ds(s * ZR, ZR)])

        @pl.when(c == 1)
        def _():
            pltpu.sync_copy(d_sh.at[pl.ds(s * ZR, ZR)],
                            d1_hbm.at[pl.ds(s * ZR, ZR)])

    return k(rowp, dep)


def _tc_body(x_ref, g0_ref, g1_ref, d0_ref, d1_ref, w1_ref, w2_ref,
             w3lo_ref, w3hi_ref, b1_ref, b2_ref, out_ref):
    xb = x_ref[...]
    hp = jnp.float32
    h1 = jnp.dot(xb, w1_ref[...], preferred_element_type=hp,
                 precision=lax.Precision.HIGHEST)
    h2 = jnp.dot(xb, w2_ref[...], preferred_element_type=hp,
                 precision=lax.Precision.HIGHEST)
    h3 = (jnp.dot(g0_ref[...], w3lo_ref[...], preferred_element_type=hp,
                  precision=lax.Precision.HIGHEST)
          + jnp.dot(g1_ref[...], w3hi_ref[...], preferred_element_type=hp,
                    precision=lax.Precision.HIGHEST))
    deg = (d0_ref[...] + d1_ref[...])[:, :1]
    out_ref[...] = (h1 + b1_ref[...]
                    + deg * (h2 + b2_ref[...])
                    - h3)


def _tc_combine(x, g0, g1, d0, d1, w1, w2, w3lo, w3hi, b1, b2):
    B = 1000
    return pl.pallas_call(
        _tc_body,
        grid=(N // B,),
        in_specs=[
            pl.BlockSpec((B, D), lambda i: (i, 0)),
            pl.BlockSpec((B, DH), lambda i: (i, 0)),
            pl.BlockSpec((B, DH), lambda i: (i, 0)),
            pl.BlockSpec((B, DH), lambda i: (i, 0)),
            pl.BlockSpec((B, DH), lambda i: (i, 0)),
            pl.BlockSpec((D, D), lambda i: (0, 0)),
            pl.BlockSpec((D, D), lambda i: (0, 0)),
            pl.BlockSpec((DH, D), lambda i: (0, 0)),
            pl.BlockSpec((DH, D), lambda i: (0, 0)),
            pl.BlockSpec((1, D), lambda i: (0, 0)),
            pl.BlockSpec((1, D), lambda i: (0, 0)),
        ],
        out_specs=pl.BlockSpec((B, D), lambda i: (i, 0)),
        out_shape=jax.ShapeDtypeStruct((N, D), jnp.float32),
    )(x, g0, g1, d0, d1, w1, w2, w3lo, w3hi, b1, b2)


def kernel(x, edge_index, self_kernel, self_bias, aggr_self_kernel,
           aggr_self_bias, aggr_neighbor_kernel):
    row = edge_index[0].astype(jnp.int32)
    col = edge_index[1].astype(jnp.int32)
    pad_n = E_PAD - E
    pad_rows = N + (jnp.arange(pad_n, dtype=jnp.int32) % 16)
    rowp = jnp.concatenate([row, pad_rows])
    colp = jnp.concatenate([col, jnp.zeros((pad_n,), jnp.int32)])

    x_lo = x[:, :DH]
    x_hi = x[:, DH:]
    g0, g1 = _sc_scatter(x_lo, x_hi, rowp, colp)
    d0, d1 = _sc_degree(rowp, g0[:8])

    out = _tc_combine(
        x, g0, g1, d0, d1,
        self_kernel, aggr_self_kernel,
        aggr_neighbor_kernel[:DH], aggr_neighbor_kernel[DH:],
        self_bias.reshape(1, D), aggr_self_bias.reshape(1, D),
    )
    return out


# speedup vs baseline: 3.8625x; 1.5083x over previous
import functools

import jax
import jax.numpy as jnp
from jax import lax
from jax.experimental import pallas as pl
from jax.experimental.pallas import tpu as pltpu
from jax.experimental.pallas import tpu_sc as plsc

N = 10000
E = 160000
D = 256
DH = 128

NC = 2
NS = 16
CH = 80
EPT = 10240
E_PAD = EPT * NS
NCHUNK = EPT // CH
NCH2 = NCHUNK // 2
DCH = 64
EPW = E_PAD // (NC * NS)
DNCHUNK = EPW // DCH
GROWS = 10240
ZR = GROWS // NS
ZB = 80


def _sc_scatter(x_lo, x_hi, rowp2, colp, zrows):
    mesh = plsc.VectorSubcoreMesh(core_axis_name="c", subcore_axis_name="s")

    @functools.partial(
        pl.kernel,
        out_type=(
            jax.ShapeDtypeStruct((GROWS, DH), jnp.float32),
            jax.ShapeDtypeStruct((GROWS, DH), jnp.float32),
        ),
        mesh=mesh,
        scratch_types=[
            pltpu.VMEM((NCHUNK, CH), jnp.int32),
            pltpu.VMEM((CH,), jnp.int32),
            pltpu.VMEM((CH,), jnp.int32),
            pltpu.VMEM((CH, DH), jnp.float32),
            pltpu.VMEM((CH, DH), jnp.float32),
            pltpu.SemaphoreType.DMA,
            pltpu.SemaphoreType.DMA,
            pltpu.SemaphoreType.DMA,
            pltpu.SemaphoreType.DMA,
            pltpu.SemaphoreType.DMA,
            pltpu.SemaphoreType.DMA,
            pltpu.VMEM_SHARED((GROWS, DH), jnp.float32),
        ],
    )
    def k(xlo_hbm, xhi_hbm, row2_hbm, col_hbm, z_hbm, g0_hbm, g1_hbm,
          row2, colv0, colv1, buf0, buf1, si0, si1, sg0, sg1, ss0, ss1, g_sh):
        c = lax.axis_index("c")
        s = lax.axis_index("s")

        pltpu.sync_copy(z_hbm, g_sh.at[pl.ds(s * ZR, ZR)])
        pltpu.sync_copy(row2_hbm.at[pl.ds(s * NCHUNK, NCHUNK)], row2)

        plsc.subcore_barrier()

        base = s * EPT

        def run(x_ref):
            def idx_start(chunk, colv, sem):
                pltpu.async_copy(col_hbm.at[pl.ds(base + chunk * CH, CH)],
                                 colv, sem)

            def idx_wait(colv, sem):
                pltpu.make_async_copy(col_hbm.at[pl.ds(base, CH)], colv,
                                      sem).wait()

            def gather_start(colv, buf, sem):
                pltpu.async_copy(x_ref.at[colv], buf, sem)

            def gather_wait(colv, buf, sem):
                pltpu.make_async_copy(x_ref.at[colv], buf, sem).wait()

            def scat_start(chunk, buf, sem):
                pltpu.async_copy(buf, g_sh.at[row2.at[chunk]], sem, add=True)

            def scat_wait(buf, sem):
                pltpu.make_async_copy(buf, g_sh.at[row2.at[0]], sem).wait()

            idx_start(0, colv0, si0)
            idx_start(1, colv1, si1)
            idx_wait(colv0, si0)
            gather_start(colv0, buf0, sg0)
            idx_wait(colv1, si1)
            gather_start(colv1, buf1, sg1)

            @pl.loop(0, NCH2)
            def _(t):
                c0 = 2 * t
                gather_wait(colv0, buf0, sg0)
                scat_start(c0, buf0, ss0)

                @pl.when(t < NCH2 - 1)
                def _():
                    idx_start(c0 + 2, colv0, si0)

                gather_wait(colv1, buf1, sg1)
                scat_start(c0 + 1, buf1, ss1)

                @pl.when(t < NCH2 - 1)
                def _():
                    idx_start(c0 + 3, colv1, si1)
                    idx_wait(colv0, si0)
                    scat_wait(buf0, ss0)
                    gather_start(colv0, buf0, sg0)
                    idx_wait(colv1, si1)
                    scat_wait(buf1, ss1)
                    gather_start(colv1, buf1, sg1)

            scat_wait(buf0, ss0)
            scat_wait(buf1, ss1)

        @pl.when(c == 0)
        def _():
            run(xlo_hbm)

        @pl.when(c == 1)
        def _():
            run(xhi_hbm)

        plsc.subcore_barrier()

        @pl.when(c == 0)
        def _():
            pltpu.sync_copy(g_sh.at[pl.ds(s * ZR, ZR)],
                            g0_hbm.at[pl.ds(s * ZR, ZR)])

        @pl.when(c == 1)
        def _():
            pltpu.sync_copy(g_sh.at[pl.ds(s * ZR, ZR)],
                            g1_hbm.at[pl.ds(s * ZR, ZR)])

    return k(x_lo, x_hi, rowp2, colp, zrows)


def _sc_degree(rowd2, zrows, dep):
    mesh = plsc.VectorSubcoreMesh(core_axis_name="c", subcore_axis_name="s")

    @functools.partial(
        pl.kernel,
        out_type=(
            jax.ShapeDtypeStruct((GROWS, DH), jnp.float32),
            jax.ShapeDtypeStruct((GROWS, DH), jnp.float32),
        ),
        mesh=mesh,
        scratch_types=[
            pltpu.VMEM((DNCHUNK, DCH), jnp.int32),
            pltpu.VMEM((DCH, DH), jnp.float32),
            pltpu.SemaphoreType.DMA,
            pltpu.VMEM_SHARED((GROWS, DH), jnp.float32),
        ],
    )
    def k(row2_hbm, z_hbm, dep_hbm, d0_hbm, d1_hbm, row2, onesb, sem, d_sh):
        del dep_hbm
        c = lax.axis_index("c")
        s = lax.axis_index("s")

        zrow = jnp.zeros((16,), jnp.float32)
        e0row = jnp.where(lax.iota(jnp.int32, 16) == 0, 1.0, 0.0)

        @pl.loop(0, DCH)
        def _(i):
            onesb[i, pl.ds(0, 16)] = e0row

            @pl.loop(1, DH // 16)
            def _(j):
                onesb[i, pl.ds(j * 16, 16)] = zrow

        pltpu.sync_copy(z_hbm, d_sh.at[pl.ds(s * ZR, ZR)])
        w = c * NS + s
        pltpu.sync_copy(row2_hbm.at[pl.ds(w * DNCHUNK, DNCHUNK)], row2)

        plsc.subcore_barrier()

        @pl.loop(0, DNCHUNK)
        def _(kk):
            pltpu.async_copy(onesb, d_sh.at[row2.at[kk]], sem, add=True)

        @pl.loop(0, DNCHUNK)
        def _(kk):
            pltpu.make_async_copy(onesb, d_sh.at[row2.at[0]], sem).wait()

        plsc.subcore_barrier()

        @pl.when(c == 0)
        def _():
            pltpu.sync_copy(d_sh.at[pl.ds(s * ZR, ZR)],
                            d0_hbm.at[pl.ds(s * ZR, ZR)])

        @pl.when(c == 1)
        def _():
            pltpu.sync_copy(d_sh.at[pl.ds(s * ZR, ZR)],
                            d1_hbm.at[pl.ds(s * ZR, ZR)])

    return k(rowd2, zrows, dep)


def _tc_body(x_ref, g0_ref, g1_ref, d0_ref, d1_ref, w1_ref, w2_ref,
             w3lo_ref, w3hi_ref, b1_ref, b2_ref, out_ref):
    xb = x_ref[...]
    hp = jnp.float32
    h1 = jnp.dot(xb, w1_ref[...], preferred_element_type=hp,
                 precision=lax.Precision.HIGHEST)
    h2 = jnp.dot(xb, w2_ref[...], preferred_element_type=hp,
                 precision=lax.Precision.HIGHEST)
    h3 = (jnp.dot(g0_ref[...], w3lo_ref[...], preferred_element_type=hp,
                  precision=lax.Precision.HIGHEST)
          + jnp.dot(g1_ref[...], w3hi_ref[...], preferred_element_type=hp,
                    precision=lax.Precision.HIGHEST))
    deg = (d0_ref[...] + d1_ref[...])[:, :1]
    out_ref[...] = (h1 + b1_ref[...]
                    + deg * (h2 + b2_ref[...])
                    - h3)


def _tc_combine(x, g0, g1, d0, d1, w1, w2, w3lo, w3hi, b1, b2):
    B = 1000
    return pl.pallas_call(
        _tc_body,
        grid=(N // B,),
        in_specs=[
            pl.BlockSpec((B, D), lambda i: (i, 0)),
            pl.BlockSpec((B, DH), lambda i: (i, 0)),
            pl.BlockSpec((B, DH), lambda i: (i, 0)),
            pl.BlockSpec((B, DH), lambda i: (i, 0)),
            pl.BlockSpec((B, DH), lambda i: (i, 0)),
            pl.BlockSpec((D, D), lambda i: (0, 0)),
            pl.BlockSpec((D, D), lambda i: (0, 0)),
            pl.BlockSpec((DH, D), lambda i: (0, 0)),
            pl.BlockSpec((DH, D), lambda i: (0, 0)),
            pl.BlockSpec((1, D), lambda i: (0, 0)),
            pl.BlockSpec((1, D), lambda i: (0, 0)),
        ],
        out_specs=pl.BlockSpec((B, D), lambda i: (i, 0)),
        out_shape=jax.ShapeDtypeStruct((N, D), jnp.float32),
    )(x, g0, g1, d0, d1, w1, w2, w3lo, w3hi, b1, b2)


def kernel(x, edge_index, self_kernel, self_bias, aggr_self_kernel,
           aggr_self_bias, aggr_neighbor_kernel):
    row = edge_index[0].astype(jnp.int32)
    col = edge_index[1].astype(jnp.int32)
    pad_n = E_PAD - E
    pad_rows = N + (jnp.arange(pad_n, dtype=jnp.int32) % 16)
    rowp = jnp.concatenate([row, pad_rows])
    colp = jnp.concatenate([col, jnp.zeros((pad_n,), jnp.int32)])

    x_lo = x[:, :DH]
    x_hi = x[:, DH:]
    rowp2 = rowp.reshape(E_PAD // CH, CH)
    rowd2 = rowp.reshape(E_PAD // DCH, DCH)
    zrows = jnp.zeros((ZR, DH), jnp.float32)
    g0, g1 = _sc_scatter(x_lo, x_hi, rowp2, colp, zrows)
    d0, d1 = _sc_degree(rowd2, zrows, g0[:8])

    out = _tc_combine(
        x, g0, g1, d0, d1,
        self_kernel, aggr_self_kernel,
        aggr_neighbor_kernel[:DH], aggr_neighbor_kernel[DH:],
        self_bias.reshape(1, D), aggr_self_bias.reshape(1, D),
    )
    return out


# speedup vs baseline: 4.2077x; 1.0894x over previous
import functools

import jax
import jax.numpy as jnp
from jax import lax
from jax.experimental import pallas as pl
from jax.experimental.pallas import tpu as pltpu
from jax.experimental.pallas import tpu_sc as plsc

N = 10000
E = 160000
D = 256
DH = 128

NC = 2
NS = 16
CH = 64
RING = 4
EPT = 10240
E_PAD = EPT * NS
NCHUNK = EPT // CH
NT = NCHUNK // RING
DCH = 64
EPW = E_PAD // (NC * NS)
DNCHUNK = EPW // DCH
GROWS = 10240
ZR = GROWS // NS
ZB = 80


def _sc_scatter(x_lo, x_hi, rowp2, colp, zrows):
    mesh = plsc.VectorSubcoreMesh(core_axis_name="c", subcore_axis_name="s")

    @functools.partial(
        pl.kernel,
        out_type=(
            jax.ShapeDtypeStruct((GROWS, DH), jnp.float32),
            jax.ShapeDtypeStruct((GROWS, DH), jnp.float32),
        ),
        mesh=mesh,
        scratch_types=(
            [pltpu.VMEM((NCHUNK // 2, CH), jnp.int32)]
            + [pltpu.VMEM((CH,), jnp.int32)] * RING
            + [pltpu.VMEM((CH, DH), jnp.float32)] * RING
            + [pltpu.SemaphoreType.DMA] * (3 * RING)
            + [pltpu.VMEM_SHARED((GROWS, DH), jnp.float32)]
        ),
    )
    def k(xlo_hbm, xhi_hbm, row2_hbm, col_hbm, z_hbm, g0_hbm, g1_hbm, row2,
          *rest):
        colv = rest[0:RING]
        buf = rest[RING:2 * RING]
        si = rest[2 * RING:3 * RING]
        sg = rest[3 * RING:4 * RING]
        ss = rest[4 * RING:5 * RING]
        g_sh = rest[5 * RING]
        c = lax.axis_index("c")
        s = lax.axis_index("s")

        pltpu.sync_copy(z_hbm, g_sh.at[pl.ds(s * ZR, ZR)])
        pltpu.sync_copy(row2_hbm.at[pl.ds(s * NCHUNK, NCHUNK // 2)], row2)

        plsc.subcore_barrier()

        base = s * EPT

        def run(x_ref):
            def idx_start(chunk, j):
                pltpu.async_copy(col_hbm.at[pl.ds(base + chunk * CH, CH)],
                                 colv[j], si[j])

            def idx_wait(j):
                pltpu.make_async_copy(col_hbm.at[pl.ds(base, CH)], colv[j],
                                      si[j]).wait()

            def gather_start(j):
                pltpu.async_copy(x_ref.at[colv[j]], buf[j], sg[j])

            def gather_wait(j):
                pltpu.make_async_copy(x_ref.at[colv[j]], buf[j], sg[j]).wait()

            def scat_start(chunk, j):
                lc = lax.rem(chunk, NCHUNK // 2)
                pltpu.async_copy(buf[j], g_sh.at[row2.at[lc]], ss[j],
                                 add=True)

            def scat_wait(j):
                pltpu.make_async_copy(buf[j], g_sh.at[row2.at[0]],
                                      ss[j]).wait()

            for j in range(RING):
                idx_start(j, j)
            for j in range(RING):
                idx_wait(j)
                gather_start(j)

            @pl.loop(0, NT)
            def _(t):
                q0 = RING * t

                @pl.when(t == NT // 2)
                def _():
                    pltpu.sync_copy(
                        row2_hbm.at[pl.ds(s * NCHUNK + NCHUNK // 2,
                                          NCHUNK // 2)], row2)

                for j in range(RING):
                    gather_wait(j)
                    scat_start(q0 + j, j)

                    @pl.when(t < NT - 1)
                    def _(j=j):
                        idx_start(q0 + j + RING, j)

                @pl.when(t < NT - 1)
                def _():
                    for j in range(RING):
                        idx_wait(j)
                        scat_wait(j)
                        gather_start(j)

            for j in range(RING):
                scat_wait(j)

        @pl.when(c == 0)
        def _():
            run(xlo_hbm)

        @pl.when(c == 1)
        def _():
            run(xhi_hbm)

        plsc.subcore_barrier()

        @pl.when(c == 0)
        def _():
            pltpu.sync_copy(g_sh.at[pl.ds(s * ZR, ZR)],
                            g0_hbm.at[pl.ds(s * ZR, ZR)])

        @pl.when(c == 1)
        def _():
            pltpu.sync_copy(g_sh.at[pl.ds(s * ZR, ZR)],
                            g1_hbm.at[pl.ds(s * ZR, ZR)])

    return k(x_lo, x_hi, rowp2, colp, zrows)


def _sc_degree(rowd2, zrows, dep):
    mesh = plsc.VectorSubcoreMesh(core_axis_name="c", subcore_axis_name="s")

    @functools.partial(
        pl.kernel,
        out_type=(
            jax.ShapeDtypeStruct((GROWS, DH), jnp.float32),
            jax.ShapeDtypeStruct((GROWS, DH), jnp.float32),
        ),
        mesh=mesh,
        scratch_types=[
            pltpu.VMEM((DNCHUNK, DCH), jnp.int32),
            pltpu.VMEM((DCH, DH), jnp.float32),
            pltpu.SemaphoreType.DMA,
            pltpu.VMEM_SHARED((GROWS, DH), jnp.float32),
        ],
    )
    def k(row2_hbm, z_hbm, dep_hbm, d0_hbm, d1_hbm, row2, onesb, sem, d_sh):
        del dep_hbm
        c = lax.axis_index("c")
        s = lax.axis_index("s")

        zrow = jnp.zeros((16,), jnp.float32)
        e0row = jnp.where(lax.iota(jnp.int32, 16) == 0, 1.0, 0.0)

        @pl.loop(0, DCH)
        def _(i):
            onesb[i, pl.ds(0, 16)] = e0row

            @pl.loop(1, DH // 16)
            def _(j):
                onesb[i, pl.ds(j * 16, 16)] = zrow

        pltpu.sync_copy(z_hbm, d_sh.at[pl.ds(s * ZR, ZR)])
        w = c * NS + s
        pltpu.sync_copy(row2_hbm.at[pl.ds(w * DNCHUNK, DNCHUNK)], row2)

        plsc.subcore_barrier()

        @pl.loop(0, DNCHUNK)
        def _(kk):
            pltpu.async_copy(onesb, d_sh.at[row2.at[kk]], sem, add=True)

        @pl.loop(0, DNCHUNK)
        def _(kk):
            pltpu.make_async_copy(onesb, d_sh.at[row2.at[0]], sem).wait()

        plsc.subcore_barrier()

        @pl.when(c == 0)
        def _():
            pltpu.sync_copy(d_sh.at[pl.ds(s * ZR, ZR)],
                            d0_hbm.at[pl.ds(s * ZR, ZR)])

        @pl.when(c == 1)
        def _():
            pltpu.sync_copy(d_sh.at[pl.ds(s * ZR, ZR)],
                            d1_hbm.at[pl.ds(s * ZR, ZR)])

    return k(rowd2, zrows, dep)


def _tc_body(x_ref, g0_ref, g1_ref, d0_ref, d1_ref, w1_ref, w2_ref,
             w3lo_ref, w3hi_ref, b1_ref, b2_ref, out_ref):
    xb = x_ref[...]
    hp = jnp.float32
    h1 = jnp.dot(xb, w1_ref[...], preferred_element_type=hp,
                 precision=lax.Precision.HIGHEST)
    h2 = jnp.dot(xb, w2_ref[...], preferred_element_type=hp,
                 precision=lax.Precision.HIGHEST)
    h3 = (jnp.dot(g0_ref[...], w3lo_ref[...], preferred_element_type=hp,
                  precision=lax.Precision.HIGHEST)
          + jnp.dot(g1_ref[...], w3hi_ref[...], preferred_element_type=hp,
                    precision=lax.Precision.HIGHEST))
    deg = (d0_ref[...] + d1_ref[...])[:, :1]
    out_ref[...] = (h1 + b1_ref[...]
                    + deg * (h2 + b2_ref[...])
                    - h3)


def _tc_combine(x, g0, g1, d0, d1, w1, w2, w3lo, w3hi, b1, b2):
    B = 1000
    return pl.pallas_call(
        _tc_body,
        grid=(N // B,),
        in_specs=[
            pl.BlockSpec((B, D), lambda i: (i, 0)),
            pl.BlockSpec((B, DH), lambda i: (i, 0)),
            pl.BlockSpec((B, DH), lambda i: (i, 0)),
            pl.BlockSpec((B, DH), lambda i: (i, 0)),
            pl.BlockSpec((B, DH), lambda i: (i, 0)),
            pl.BlockSpec((D, D), lambda i: (0, 0)),
            pl.BlockSpec((D, D), lambda i: (0, 0)),
            pl.BlockSpec((DH, D), lambda i: (0, 0)),
            pl.BlockSpec((DH, D), lambda i: (0, 0)),
            pl.BlockSpec((1, D), lambda i: (0, 0)),
            pl.BlockSpec((1, D), lambda i: (0, 0)),
        ],
        out_specs=pl.BlockSpec((B, D), lambda i: (i, 0)),
        out_shape=jax.ShapeDtypeStruct((N, D), jnp.float32),
    )(x, g0, g1, d0, d1, w1, w2, w3lo, w3hi, b1, b2)


def kernel(x, edge_index, self_kernel, self_bias, aggr_self_kernel,
           aggr_self_bias, aggr_neighbor_kernel):
    row = edge_index[0].astype(jnp.int32)
    col = edge_index[1].astype(jnp.int32)
    pad_n = E_PAD - E
    pad_rows = N + (jnp.arange(pad_n, dtype=jnp.int32) % 16)
    rowp = jnp.concatenate([row, pad_rows])
    colp = jnp.concatenate([col, jnp.zeros((pad_n,), jnp.int32)])

    x_lo = x[:, :DH]
    x_hi = x[:, DH:]
    rowp2 = rowp.reshape(E_PAD // CH, CH)
    rowd2 = rowp.reshape(E_PAD // DCH, DCH)
    zrows = jnp.zeros((ZR, DH), jnp.float32)
    g0, g1 = _sc_scatter(x_lo, x_hi, rowp2, colp, zrows)
    d0, d1 = _sc_degree(rowd2, zrows, g0[:8])

    out = _tc_combine(
        x, g0, g1, d0, d1,
        self_kernel, aggr_self_kernel,
        aggr_neighbor_kernel[:DH], aggr_neighbor_kernel[DH:],
        self_bias.reshape(1, D), aggr_self_bias.reshape(1, D),
    )
    return out


# speedup vs baseline: 4.2808x; 1.0174x over previous
import functools

import jax
import jax.numpy as jnp
from jax import lax
from jax.experimental import pallas as pl
from jax.experimental.pallas import tpu as pltpu
from jax.experimental.pallas import tpu_sc as plsc

N = 10000
E = 160000
D = 256
DH = 128

NC = 2
NS = 16
CH = 64
RING = 4
EPT = 10240
E_PAD = EPT * NS
NCHUNK = EPT // CH
NT = NCHUNK // RING
DCH = 64
EPW = E_PAD // (NC * NS)
DNCHUNK = EPW // DCH
GROWS = 10240
ZR = GROWS // NS
ZB = 80


def _sc_scatter(x_lo, x_hi, rowp2, colp, zrows):
    mesh = plsc.VectorSubcoreMesh(core_axis_name="c", subcore_axis_name="s")

    @functools.partial(
        pl.kernel,
        out_type=(
            jax.ShapeDtypeStruct((GROWS, DH), jnp.float32),
            jax.ShapeDtypeStruct((GROWS, DH), jnp.float32),
        ),
        mesh=mesh,
        scratch_types=(
            [pltpu.VMEM((NCHUNK // 2, CH), jnp.int32)]
            + [pltpu.VMEM((CH,), jnp.int32)] * RING
            + [pltpu.VMEM((CH, DH), jnp.float32)] * RING
            + [pltpu.SemaphoreType.DMA] * (3 * RING)
            + [pltpu.VMEM_SHARED((GROWS, DH), jnp.float32)]
        ),
    )
    def k(xlo_hbm, xhi_hbm, row2_hbm, col_hbm, z_hbm, g0_hbm, g1_hbm, row2,
          *rest):
        colv = rest[0:RING]
        buf = rest[RING:2 * RING]
        si = rest[2 * RING:3 * RING]
        sg = rest[3 * RING:4 * RING]
        ss = rest[4 * RING:5 * RING]
        g_sh = rest[5 * RING]
        c = lax.axis_index("c")
        s = lax.axis_index("s")

        pltpu.sync_copy(z_hbm, g_sh.at[pl.ds(s * ZR, ZR)])
        pltpu.sync_copy(row2_hbm.at[pl.ds(s * NCHUNK, NCHUNK // 2)], row2)

        plsc.subcore_barrier()

        base = s * EPT

        def run(x_ref):
            def idx_start(chunk, j):
                pltpu.async_copy(col_hbm.at[pl.ds(base + chunk * CH, CH)],
                                 colv[j], si[j])

            def idx_wait(j):
                pltpu.make_async_copy(col_hbm.at[pl.ds(base, CH)], colv[j],
                                      si[j]).wait()

            def gather_start(j):
                pltpu.async_copy(x_ref.at[colv[j]], buf[j], sg[j])

            def gather_wait(j):
                pltpu.make_async_copy(x_ref.at[colv[j]], buf[j], sg[j]).wait()

            def scat_start(chunk, j):
                lc = lax.rem(chunk, NCHUNK // 2)
                pltpu.async_copy(buf[j], g_sh.at[row2.at[lc]], ss[j],
                                 add=True)

            def scat_wait(j):
                pltpu.make_async_copy(buf[j], g_sh.at[row2.at[0]],
                                      ss[j]).wait()

            for j in range(RING):
                idx_start(j, j)
            for j in range(RING):
                idx_wait(j)
                gather_start(j)

            @pl.loop(0, NT)
            def _(t):
                q0 = RING * t

                @pl.when(t == NT // 2)
                def _():
                    pltpu.sync_copy(
                        row2_hbm.at[pl.ds(s * NCHUNK + NCHUNK // 2,
                                          NCHUNK // 2)], row2)

                for j in range(RING):
                    gather_wait(j)
                    scat_start(q0 + j, j)

                    @pl.when(t < NT - 1)
                    def _(j=j):
                        idx_start(q0 + j + RING, j)

                @pl.when(t < NT - 1)
                def _():
                    for j in range(RING):
                        idx_wait(j)
                        scat_wait(j)
                        gather_start(j)

            for j in range(RING):
                scat_wait(j)

        @pl.when(c == 0)
        def _():
            run(xlo_hbm)

        @pl.when(c == 1)
        def _():
            run(xhi_hbm)

        plsc.subcore_barrier()

        @pl.when(c == 0)
        def _():
            pltpu.sync_copy(g_sh.at[pl.ds(s * ZR, ZR)],
                            g0_hbm.at[pl.ds(s * ZR, ZR)])

        @pl.when(c == 1)
        def _():
            pltpu.sync_copy(g_sh.at[pl.ds(s * ZR, ZR)],
                            g1_hbm.at[pl.ds(s * ZR, ZR)])

    return k(x_lo, x_hi, rowp2, colp, zrows)


def _sc_degree(rowd2, zrows, dep):
    mesh = plsc.VectorSubcoreMesh(core_axis_name="c", subcore_axis_name="s")

    @functools.partial(
        pl.kernel,
        out_type=(
            jax.ShapeDtypeStruct((GROWS, DH), jnp.float32),
            jax.ShapeDtypeStruct((GROWS, DH), jnp.float32),
        ),
        mesh=mesh,
        scratch_types=[
            pltpu.VMEM((DNCHUNK, DCH), jnp.int32),
            pltpu.VMEM((DCH, DH), jnp.float32),
            pltpu.SemaphoreType.DMA,
            pltpu.VMEM_SHARED((GROWS, DH), jnp.float32),
        ],
    )
    def k(row2_hbm, z_hbm, dep_hbm, d0_hbm, d1_hbm, row2, onesb, sem, d_sh):
        del dep_hbm
        c = lax.axis_index("c")
        s = lax.axis_index("s")

        zrow = jnp.zeros((16,), jnp.float32)
        e0row = jnp.where(lax.iota(jnp.int32, 16) == 0, 1.0, 0.0)

        @pl.loop(0, DCH)
        def _(i):
            onesb[i, pl.ds(0, 16)] = e0row

            @pl.loop(1, DH // 16)
            def _(j):
                onesb[i, pl.ds(j * 16, 16)] = zrow

        pltpu.sync_copy(z_hbm, d_sh.at[pl.ds(s * ZR, ZR)])
        w = c * NS + s
        pltpu.sync_copy(row2_hbm.at[pl.ds(w * DNCHUNK, DNCHUNK)], row2)

        plsc.subcore_barrier()

        @pl.loop(0, DNCHUNK)
        def _(kk):
            pltpu.async_copy(onesb, d_sh.at[row2.at[kk]], sem, add=True)

        @pl.loop(0, DNCHUNK)
        def _(kk):
            pltpu.make_async_copy(onesb, d_sh.at[row2.at[0]], sem).wait()

        plsc.subcore_barrier()

        @pl.when(c == 0)
        def _():
            pltpu.sync_copy(d_sh.at[pl.ds(s * ZR, ZR)],
                            d0_hbm.at[pl.ds(s * ZR, ZR)])

        @pl.when(c == 1)
        def _():
            pltpu.sync_copy(d_sh.at[pl.ds(s * ZR, ZR)],
                            d1_hbm.at[pl.ds(s * ZR, ZR)])

    return k(rowd2, zrows, dep)


def _tc_xw_body(x_ref, w1_ref, w2_ref, b1_ref, b2_ref, h1_ref, h2_ref):
    xb = x_ref[...]
    hp = jnp.float32
    h1_ref[...] = jnp.dot(xb, w1_ref[...], preferred_element_type=hp,
                          precision=lax.Precision.HIGHEST) + b1_ref[...]
    h2_ref[...] = jnp.dot(xb, w2_ref[...], preferred_element_type=hp,
                          precision=lax.Precision.HIGHEST) + b2_ref[...]


def _tc_xw(x, w1, w2, b1, b2):
    B = 1000
    return pl.pallas_call(
        _tc_xw_body,
        grid=(N // B,),
        in_specs=[
            pl.BlockSpec((B, D), lambda i: (i, 0)),
            pl.BlockSpec((D, D), lambda i: (0, 0)),
            pl.BlockSpec((D, D), lambda i: (0, 0)),
            pl.BlockSpec((1, D), lambda i: (0, 0)),
            pl.BlockSpec((1, D), lambda i: (0, 0)),
        ],
        out_specs=(pl.BlockSpec((B, D), lambda i: (i, 0)),
                   pl.BlockSpec((B, D), lambda i: (i, 0))),
        out_shape=(jax.ShapeDtypeStruct((N, D), jnp.float32),
                   jax.ShapeDtypeStruct((N, D), jnp.float32)),
    )(x, w1, w2, b1, b2)


def _tc_body(h1_ref, h2_ref, g0_ref, g1_ref, d0_ref, d1_ref,
             w3lo_ref, w3hi_ref, out_ref):
    hp = jnp.float32
    h3 = (jnp.dot(g0_ref[...], w3lo_ref[...], preferred_element_type=hp,
                  precision=lax.Precision.HIGHEST)
          + jnp.dot(g1_ref[...], w3hi_ref[...], preferred_element_type=hp,
                    precision=lax.Precision.HIGHEST))
    deg = (d0_ref[...] + d1_ref[...])[:, :1]
    out_ref[...] = h1_ref[...] + deg * h2_ref[...] - h3


def _tc_combine(h1, h2, g0, g1, d0, d1, w3lo, w3hi):
    B = 1000
    return pl.pallas_call(
        _tc_body,
        grid=(N // B,),
        in_specs=[
            pl.BlockSpec((B, D), lambda i: (i, 0)),
            pl.BlockSpec((B, D), lambda i: (i, 0)),
            pl.BlockSpec((B, DH), lambda i: (i, 0)),
            pl.BlockSpec((B, DH), lambda i: (i, 0)),
            pl.BlockSpec((B, DH), lambda i: (i, 0)),
            pl.BlockSpec((B, DH), lambda i: (i, 0)),
            pl.BlockSpec((DH, D), lambda i: (0, 0)),
            pl.BlockSpec((DH, D), lambda i: (0, 0)),
        ],
        out_specs=pl.BlockSpec((B, D), lambda i: (i, 0)),
        out_shape=jax.ShapeDtypeStruct((N, D), jnp.float32),
    )(h1, h2, g0, g1, d0, d1, w3lo, w3hi)


def kernel(x, edge_index, self_kernel, self_bias, aggr_self_kernel,
           aggr_self_bias, aggr_neighbor_kernel):
    row = edge_index[0].astype(jnp.int32)
    col = edge_index[1].astype(jnp.int32)
    pad_n = E_PAD - E
    pad_rows = N + (jnp.arange(pad_n, dtype=jnp.int32) % 16)
    rowp = jnp.concatenate([row, pad_rows])
    colp = jnp.concatenate([col, jnp.zeros((pad_n,), jnp.int32)])

    x_lo = x[:, :DH]
    x_hi = x[:, DH:]
    rowp2 = rowp.reshape(E_PAD // CH, CH)
    rowd2 = rowp.reshape(E_PAD // DCH, DCH)
    zrows = jnp.zeros((ZR, DH), jnp.float32)
    g0, g1 = _sc_scatter(x_lo, x_hi, rowp2, colp, zrows)
    d0, d1 = _sc_degree(rowd2, zrows, g0[:8])
    h1, h2 = _tc_xw(x, self_kernel, aggr_self_kernel,
                    self_bias.reshape(1, D), aggr_self_bias.reshape(1, D))

    out = _tc_combine(h1, h2, g0, g1, d0, d1,
                      aggr_neighbor_kernel[:DH], aggr_neighbor_kernel[DH:])
    return out
